# initial kernel scaffold (unmeasured)
import jax
import jax.numpy as jnp
from jax import lax
from jax.experimental import pallas as pl
from jax.experimental.pallas import tpu as pltpu


def kernel(
    x,
):
    def body(*refs):
        pass

    out_shape = jax.ShapeDtypeStruct(..., jnp.float32)
    return pl.pallas_call(body, out_shape=out_shape)(...)


# baseline (device time: 9911 ns/iter reference)
import jax
import jax.numpy as jnp
from jax import lax
from jax.experimental import pallas as pl
from jax.experimental.pallas import tpu as pltpu

N_DEV = 4


def kernel(x):
    m, n = x.shape

    def body(x_ref, out_ref, pfx_ref, send_ref, send_sem, recv_sem):
        my = lax.axis_index("i")

        acc = x_ref[:, :]
        d = 1
        while d < m:
            shifted = jnp.concatenate(
                [jnp.ones((d, n), jnp.float32), acc[: m - d, :]], axis=0
            )
            acc = acc * shifted
            d *= 2

        total = acc[m - 1 : m, :]

        @pl.when(my == 0)
        def _():
            pfx_ref[:, :] = jnp.ones((1, n), jnp.float32)

        @pl.when(my > 0)
        def _():
            recv = pltpu.make_async_remote_copy(
                src_ref=send_ref,
                dst_ref=pfx_ref,
                send_sem=send_sem,
                recv_sem=recv_sem,
                device_id=((my + N_DEV - 1) % N_DEV,),
                device_id_type=pl.DeviceIdType.MESH,
            )
            recv.wait_recv()

        @pl.when(my < N_DEV - 1)
        def _():
            send_ref[:, :] = pfx_ref[:, :] * total
            send = pltpu.make_async_remote_copy(
                src_ref=send_ref,
                dst_ref=pfx_ref,
                send_sem=send_sem,
                recv_sem=recv_sem,
                device_id=(my + 1,),
                device_id_type=pl.DeviceIdType.MESH,
            )
            send.start()
            send.wait_send()

        out_ref[:, :] = acc * pfx_ref[0:1, :]

    return pl.pallas_call(
        body,
        out_shape=jax.ShapeDtypeStruct((m, n), jnp.float32),
        in_specs=[pl.BlockSpec(memory_space=pltpu.VMEM)],
        out_specs=pl.BlockSpec(memory_space=pltpu.VMEM),
        scratch_shapes=[
            pltpu.VMEM((1, n), jnp.float32),
            pltpu.VMEM((1, n), jnp.float32),
            pltpu.SemaphoreType.DMA,
            pltpu.SemaphoreType.DMA,
        ],
    )(x)


# device time: 6774 ns/iter; 1.4631x vs baseline; 1.4631x over previous
import jax
import jax.numpy as jnp
from jax import lax
from jax.experimental import pallas as pl
from jax.experimental.pallas import tpu as pltpu

N_DEV = 4


def kernel(x):
    m, n = x.shape

    def body(x_ref, out_ref, totals_ref, send_ref, send_sems, recv_sems):
        my = lax.axis_index("i")
        peers = [lax.rem(my + k, N_DEV) for k in range(1, N_DEV)]

        barrier = pltpu.get_barrier_semaphore()
        for p in peers:
            pl.semaphore_signal(
                barrier, inc=1, device_id=(p,),
                device_id_type=pl.DeviceIdType.MESH,
            )
        pl.semaphore_wait(barrier, N_DEV - 1)

        acc = x_ref[:, :]
        d = 1
        while d < m:
            shifted = jnp.concatenate(
                [jnp.ones((d, n), jnp.float32), acc[: m - d, :]], axis=0
            )
            acc = acc * shifted
            d *= 2

        send_ref[:, :] = acc[m - 1 : m, :]
        sends = []
        for k, p in enumerate(peers):
            rdma = pltpu.make_async_remote_copy(
                src_ref=send_ref,
                dst_ref=totals_ref.at[my],
                send_sem=send_sems.at[k],
                recv_sem=recv_sems.at[my],
                device_id=(p,),
                device_id_type=pl.DeviceIdType.MESH,
            )
            rdma.start()
            sends.append(rdma)

        for p in peers:
            recv = pltpu.make_async_remote_copy(
                src_ref=send_ref,
                dst_ref=totals_ref.at[p],
                send_sem=send_sems.at[0],
                recv_sem=recv_sems.at[p],
                device_id=(p,),
                device_id_type=pl.DeviceIdType.MESH,
            )
            recv.wait_recv()

        row_ids = lax.broadcasted_iota(jnp.int32, (N_DEV, 1, n), 0)
        masked = jnp.where(row_ids < my, totals_ref[:, :, :], 1.0)
        pfx = masked[0]
        for r in range(1, N_DEV):
            pfx = pfx * masked[r]

        out_ref[:, :] = acc * pfx

        for rdma in sends:
            rdma.wait_send()

    return pl.pallas_call(
        body,
        out_shape=jax.ShapeDtypeStruct((m, n), jnp.float32),
        in_specs=[pl.BlockSpec(memory_space=pltpu.VMEM)],
        out_specs=pl.BlockSpec(memory_space=pltpu.VMEM),
        scratch_shapes=[
            pltpu.VMEM((N_DEV, 1, n), jnp.float32),
            pltpu.VMEM((1, n), jnp.float32),
            pltpu.SemaphoreType.DMA((N_DEV - 1,)),
            pltpu.SemaphoreType.DMA((N_DEV,)),
        ],
        compiler_params=pltpu.CompilerParams(collective_id=0),
    )(x)


# device time: 6627 ns/iter; 1.4955x vs baseline; 1.0222x over previous
import jax
import jax.numpy as jnp
from jax import lax
from jax.experimental import pallas as pl
from jax.experimental.pallas import tpu as pltpu

N_DEV = 4


def kernel(x):
    m, n = x.shape

    def body(x_ref, out_ref, totals_ref, send_ref, send_sems, recv_sems):
        my = lax.axis_index("i")
        peers = [lax.rem(my + k, N_DEV) for k in range(1, N_DEV)]

        barrier = pltpu.get_barrier_semaphore()
        for p in peers:
            pl.semaphore_signal(
                barrier, inc=1, device_id=(p,),
                device_id_type=pl.DeviceIdType.MESH,
            )

        t = x_ref[:, :]
        size = m
        while size > 1:
            half = size // 2
            t = t[:half, :] * t[half:size, :]
            size = half
        send_ref[:, :] = t

        pl.semaphore_wait(barrier, N_DEV - 1)

        sends = []
        for k, p in enumerate(peers):
            rdma = pltpu.make_async_remote_copy(
                src_ref=send_ref,
                dst_ref=totals_ref.at[my],
                send_sem=send_sems.at[k],
                recv_sem=recv_sems.at[my],
                device_id=(p,),
                device_id_type=pl.DeviceIdType.MESH,
            )
            rdma.start()
            sends.append(rdma)

        acc = x_ref[:, :]
        d = 1
        while d < m:
            shifted = jnp.concatenate(
                [jnp.ones((d, n), jnp.float32), acc[: m - d, :]], axis=0
            )
            acc = acc * shifted
            d *= 2

        for p in peers:
            recv = pltpu.make_async_remote_copy(
                src_ref=send_ref,
                dst_ref=totals_ref.at[p],
                send_sem=send_sems.at[0],
                recv_sem=recv_sems.at[p],
                device_id=(p,),
                device_id_type=pl.DeviceIdType.MESH,
            )
            recv.wait_recv()

        row_ids = lax.broadcasted_iota(jnp.int32, (N_DEV, 1, n), 0)
        masked = jnp.where(row_ids < my, totals_ref[:, :, :], 1.0)
        pfx = masked[0]
        for r in range(1, N_DEV):
            pfx = pfx * masked[r]

        out_ref[:, :] = acc * pfx

        for rdma in sends:
            rdma.wait_send()

    return pl.pallas_call(
        body,
        out_shape=jax.ShapeDtypeStruct((m, n), jnp.float32),
        in_specs=[pl.BlockSpec(memory_space=pltpu.VMEM)],
        out_specs=pl.BlockSpec(memory_space=pltpu.VMEM),
        scratch_shapes=[
            pltpu.VMEM((N_DEV, 1, n), jnp.float32),
            pltpu.VMEM((1, n), jnp.float32),
            pltpu.SemaphoreType.DMA((N_DEV - 1,)),
            pltpu.SemaphoreType.DMA((N_DEV,)),
        ],
        compiler_params=pltpu.CompilerParams(collective_id=0),
    )(x)


# device time: 6595 ns/iter; 1.5028x vs baseline; 1.0049x over previous
import jax
import jax.numpy as jnp
from jax import lax
from jax.experimental import pallas as pl
from jax.experimental.pallas import tpu as pltpu

N_DEV = 4


def kernel(x):
    m, n = x.shape

    def body(x_ref, out_ref, totals_ref, send_ref, send_sems, recv_sems):
        my = lax.axis_index("i")
        peers = [lax.rem(my + k, N_DEV) for k in range(1, N_DEV)]

        barrier = pltpu.get_barrier_semaphore()
        for p in peers:
            pl.semaphore_signal(
                barrier, inc=1, device_id=(p,),
                device_id_type=pl.DeviceIdType.MESH,
            )

        t = x_ref[:, :]
        size = m
        while size > 1:
            half = size // 2
            t = t[:half, :] * t[half:size, :]
            size = half
        send_ref[:, :] = t

        pl.semaphore_wait(barrier, N_DEV - 1)

        sends = []
        for k, p in enumerate(peers):
            rdma = pltpu.make_async_remote_copy(
                src_ref=send_ref,
                dst_ref=totals_ref.at[my],
                send_sem=send_sems.at[k],
                recv_sem=recv_sems.at[my],
                device_id=(p,),
                device_id_type=pl.DeviceIdType.MESH,
            )
            rdma.start()
            sends.append(rdma)

        acc = x_ref[:, :]
        d = 1
        while d < m:
            shifted = jnp.concatenate(
                [jnp.ones((d, n), jnp.float32), acc[: m - d, :]], axis=0
            )
            acc = acc * shifted
            d *= 2

        for p in peers:
            recv = pltpu.make_async_remote_copy(
                src_ref=send_ref,
                dst_ref=totals_ref.at[p],
                send_sem=send_sems.at[0],
                recv_sem=recv_sems.at[p],
                device_id=(p,),
                device_id_type=pl.DeviceIdType.MESH,
            )
            recv.wait_recv()

        row_ids = lax.broadcasted_iota(jnp.int32, (N_DEV, 1, n), 0)
        masked = jnp.where(row_ids < my, totals_ref[:, :, :], 1.0)
        pfx = masked[0]
        for r in range(1, N_DEV):
            pfx = pfx * masked[r]

        out_ref[:, :] = acc * pfx

        for rdma in sends:
            rdma.wait_send()

    def body_compute_only(x_ref, out_ref, totals_ref, send_ref, send_sems, recv_sems):
        t = x_ref[:, :]
        size = m
        while size > 1:
            half = size // 2
            t = t[:half, :] * t[half:size, :]
            size = half
        send_ref[:, :] = t
        acc = x_ref[:, :]
        d = 1
        while d < m:
            shifted = jnp.concatenate(
                [jnp.ones((d, n), jnp.float32), acc[: m - d, :]], axis=0
            )
            acc = acc * shifted
            d *= 2
        out_ref[:, :] = acc * send_ref[0:1, :]

    def body_comm_only(x_ref, out_ref, totals_ref, send_ref, send_sems, recv_sems):
        my = lax.axis_index("i")
        peers = [lax.rem(my + k, N_DEV) for k in range(1, N_DEV)]
        barrier = pltpu.get_barrier_semaphore()
        for p in peers:
            pl.semaphore_signal(
                barrier, inc=1, device_id=(p,),
                device_id_type=pl.DeviceIdType.MESH,
            )
        t = x_ref[:, :]
        size = m
        while size > 1:
            half = size // 2
            t = t[:half, :] * t[half:size, :]
            size = half
        send_ref[:, :] = t
        pl.semaphore_wait(barrier, N_DEV - 1)
        sends = []
        for k, p in enumerate(peers):
            rdma = pltpu.make_async_remote_copy(
                src_ref=send_ref,
                dst_ref=totals_ref.at[my],
                send_sem=send_sems.at[k],
                recv_sem=recv_sems.at[my],
                device_id=(p,),
                device_id_type=pl.DeviceIdType.MESH,
            )
            rdma.start()
            sends.append(rdma)
        for p in peers:
            recv = pltpu.make_async_remote_copy(
                src_ref=send_ref,
                dst_ref=totals_ref.at[p],
                send_sem=send_sems.at[0],
                recv_sem=recv_sems.at[p],
                device_id=(p,),
                device_id_type=pl.DeviceIdType.MESH,
            )
            recv.wait_recv()
        row_ids = lax.broadcasted_iota(jnp.int32, (N_DEV, 1, n), 0)
        masked = jnp.where(row_ids < my, totals_ref[:, :, :], 1.0)
        pfx = masked[0]
        for r in range(1, N_DEV):
            pfx = pfx * masked[r]
        out_ref[:, :] = x_ref[:, :] * pfx
        for rdma in sends:
            rdma.wait_send()

    import os
    _probe_path = os.path.join(os.path.dirname(__file__), "probe.txt")
    _probe = ""
    if os.path.exists(_probe_path):
        _probe = open(_probe_path).read().strip()
    if _probe == "compute":
        body = body_compute_only
    elif _probe == "comm":
        body = body_comm_only

    return pl.pallas_call(
        body,
        out_shape=jax.ShapeDtypeStruct((m, n), jnp.float32),
        in_specs=[pl.BlockSpec(memory_space=pltpu.VMEM)],
        out_specs=pl.BlockSpec(memory_space=pltpu.VMEM),
        scratch_shapes=[
            pltpu.VMEM((N_DEV, 1, n), jnp.float32),
            pltpu.VMEM((1, n), jnp.float32),
            pltpu.SemaphoreType.DMA((N_DEV - 1,)),
            pltpu.SemaphoreType.DMA((N_DEV,)),
        ],
        **(
            {}
            if _probe == "compute"
            else {"compiler_params": pltpu.CompilerParams(collective_id=0)}
        ),
    )(x)
